# baseline (speedup 1.0000x reference)
import functools

import jax
import jax.numpy as jnp
from jax import lax
from jax.experimental import pallas as pl
from jax.experimental.pallas import tpu as pltpu
from jax.experimental.pallas import tpu_sc as plsc

R = 128
C = 32768
L = 16
NC = 2
NS = 16
NW = NC * NS
ROWS_PER_W = R // NW
U = 8
STEPS = C // L

_mesh = plsc.VectorSubcoreMesh(core_axis_name="c", subcore_axis_name="s")


def _shuffle(x, idx):
    return lax.gather(
        x, idx[:, None],
        lax.GatherDimensionNumbers(
            offset_dims=(), collapsed_slice_dims=(0,), start_index_map=(0,)),
        slice_sizes=(1,),
        mode=lax.GatherScatterMode.PROMISE_IN_BOUNDS)


@functools.partial(
    pl.kernel,
    out_type=jax.ShapeDtypeStruct((R, C), jnp.float32),
    mesh=_mesh,
    scratch_types=[
        pltpu.VMEM((C,), jnp.float32),
        pltpu.VMEM((C,), jnp.float32),
        pltpu.VMEM((C,), jnp.float32),
        pltpu.VMEM((L,), jnp.float32),
        pltpu.VMEM((L,), jnp.int32),
        pltpu.SemaphoreType.DMA,
        pltpu.SemaphoreType.DMA,
        pltpu.SemaphoreType.DMA,
        pltpu.SemaphoreType.DMA,
    ],
    compiler_params=pltpu.CompilerParams(needs_layout_passes=False),
)
def _argmax_onehot(data_hbm, out_hbm, in0, in1, zero_v, patch_v, idx_v,
                   sem0, sem1, sem_out, sem_patch):
    wid = lax.axis_index("s") * NC + lax.axis_index("c")
    lanes = lax.iota(jnp.int32, L)
    zeros = jnp.zeros((L,), jnp.float32)
    bufs = (in0, in1)
    sems = (sem0, sem1)
    base_row = wid * ROWS_PER_W

    cps = [pltpu.async_copy(data_hbm.at[base_row], in0, sem0),
           pltpu.async_copy(data_hbm.at[base_row + 1], in1, sem1)]

    def zfill(t, _):
        base = t * (U * L)
        for k in range(U):
            zero_v[pl.ds(base + k * L, L)] = zeros
        return 0

    lax.fori_loop(0, STEPS // U, zfill, 0)

    out_cps = [
        pltpu.async_copy(zero_v, out_hbm.at[base_row + r], sem_out)
        for r in range(ROWS_PER_W)
    ]

    row_idx = []
    for r in range(ROWS_PER_W):
        cps[r % 2].wait()
        buf = bufs[r % 2]

        def step(t, carry, buf=buf):
            bv, bi = carry
            base = t * (U * L)
            for k in range(U):
                v = buf[pl.ds(base + k * L, L)]
                idx = (base + k * L) + lanes
                upd = v > bv
                bv = jnp.where(upd, v, bv)
                bi = jnp.where(upd, idx, bi)
            return bv, bi

        init = (jnp.full((L,), -jnp.inf, jnp.float32),
                jnp.zeros((L,), jnp.int32))
        bv, bi = lax.fori_loop(0, STEPS // U, step, init)

        for k in (8, 4, 2, 1):
            pv = _shuffle(bv, lanes ^ k)
            pi = _shuffle(bi, lanes ^ k)
            take = (pv > bv) | ((pv == bv) & (pi < bi))
            bv = jnp.where(take, pv, bv)
            bi = jnp.where(take, pi, bi)

        row_idx.append(bi[0])
        if r + 2 < ROWS_PER_W:
            cps[r % 2] = pltpu.async_copy(
                data_hbm.at[base_row + r + 2], bufs[r % 2], sems[r % 2])

    for cp in out_cps:
        cp.wait()
    for r in range(ROWS_PER_W):
        s = row_idx[r]
        patch_v[...] = jnp.where(lanes == s % L, 1.0, 0.0).astype(jnp.float32)
        col0 = (s // L) * L
        pltpu.async_copy(patch_v, out_hbm.at[base_row + r, pl.ds(col0, L)],
                         sem_patch).wait()


def kernel(data):
    return _argmax_onehot(data)
